# Initial kernel scaffold; baseline (speedup 1.0000x reference)
import jax
import jax.numpy as jnp
from jax.experimental import pallas as pl


def kernel(preds, gts):
    raise NotImplementedError("write your pallas kernel here")


# speedup vs baseline: 2.4501x; 2.4501x over previous
import jax
import jax.numpy as jnp
from jax import lax
from jax.experimental import pallas as pl
from jax.experimental.pallas import tpu as pltpu
from jax.experimental.pallas import tpu_sc as plsc

B, N, P, D = 64, 16384, 64, 34
L = 16
NC, NS = 2, 16
NW = NC * NS
IPW = B // NW
PG = P // L
FLAT = P * D * 3


def _image_loss(gts_v, preds_v, lane):
    err = jnp.zeros((L,), jnp.float32)
    npeople = jnp.zeros((L,), jnp.int32)
    for g in range(PG):
        pbase = (g * L + lane) * (D * 3)

        def body(d, carry):
            e, vacc = carry
            b = pbase + d * 3
            val = plsc.load_gather(gts_v, [b])
            pif = plsc.load_gather(gts_v, [b + 1])
            flg = plsc.load_gather(gts_v, [b + 2])
            prd = plsc.load_gather(preds_v, [pif.astype(jnp.int32)])
            m = flg > 0.0
            e = e + jnp.where(m, jnp.abs(prd - val), 0.0)
            vacc = jnp.maximum(vacc, m.astype(jnp.float32))
            return e, vacc

        err, vacc = lax.fori_loop(
            0, D, body, (err, jnp.zeros((L,), jnp.float32)))
        npeople = npeople + plsc.all_reduce_population_count(vacc > 0.0)
    total = jnp.sum(err)
    return total / jnp.maximum(npeople.astype(jnp.float32), 1.0)


def _body(preds_hbm, gts_hbm, out_hbm,
          preds_v0, gts_v0, preds_v1, gts_v1, res_v, sem):
    wid = lax.axis_index("s") * NC + lax.axis_index("c")
    i0 = wid * IPW
    lane = lax.iota(jnp.int32, L)
    cp0 = pltpu.async_copy(preds_hbm.at[i0], preds_v0, sem)
    cg0 = pltpu.async_copy(gts_hbm.at[i0], gts_v0, sem)
    cp1 = pltpu.async_copy(preds_hbm.at[i0 + 1], preds_v1, sem)
    cg1 = pltpu.async_copy(gts_hbm.at[i0 + 1], gts_v1, sem)
    cp0.wait()
    cg0.wait()
    loss0 = _image_loss(gts_v0, preds_v0, lane)
    cp1.wait()
    cg1.wait()
    loss1 = _image_loss(gts_v1, preds_v1, lane)
    res = jnp.where(lane == 0, loss0, jnp.where(lane == 1, loss1, 0.0))
    res_v[...] = res
    pltpu.sync_copy(res_v, out_hbm.at[wid])


def kernel(preds, gts):
    gts_flat = gts.reshape(B, FLAT)
    mesh = plsc.VectorSubcoreMesh(core_axis_name="c", subcore_axis_name="s")
    f = pl.kernel(
        _body,
        mesh=mesh,
        out_type=jax.ShapeDtypeStruct((NW, L), jnp.float32),
        scratch_types=[
            pltpu.VMEM((N,), jnp.float32),
            pltpu.VMEM((FLAT,), jnp.float32),
            pltpu.VMEM((N,), jnp.float32),
            pltpu.VMEM((FLAT,), jnp.float32),
            pltpu.VMEM((L,), jnp.float32),
            pltpu.SemaphoreType.DMA,
        ],
        compiler_params=pltpu.CompilerParams(needs_layout_passes=False),
    )
    out2d = f(preds, gts_flat)
    return out2d[:, :IPW].reshape(B)


# speedup vs baseline: 2.4743x; 1.0099x over previous
import jax
import jax.numpy as jnp
from jax import lax
from jax.experimental import pallas as pl
from jax.experimental.pallas import tpu as pltpu
from jax.experimental.pallas import tpu_sc as plsc

B, N, P, D = 64, 16384, 64, 34
L = 16
NC, NS = 2, 16
NW = NC * NS
IPW = B // NW
PG = P // L
FLAT = P * D * 3


def _image_loss(gts_v, preds_v, lane):
    zero = jnp.zeros((L,), jnp.float32)
    init = (
        tuple(zero for _ in range(PG)),
        tuple(zero for _ in range(PG)),
        tuple((g * L + lane) * (D * 3) for g in range(PG)),
    )

    @plsc.parallel_loop(0, D, carry=init, unroll=2)
    def final(d, carry):
        errs, vaccs, bs = carry
        ne, nv, nb = [], [], []
        for g in range(PG):
            b = bs[g]
            val = plsc.load_gather(gts_v, [b])
            pif = plsc.load_gather(gts_v, [b + 1])
            flg = plsc.load_gather(gts_v, [b + 2])
            prd = plsc.load_gather(preds_v, [pif.astype(jnp.int32)])
            m = flg > 0.0
            ne.append(errs[g] + jnp.where(m, jnp.abs(prd - val), 0.0))
            nv.append(jnp.maximum(vaccs[g], m.astype(jnp.float32)))
            nb.append(b + 3)
        return tuple(ne), tuple(nv), tuple(nb)

    errs, vaccs, _ = final
    npeople = jnp.zeros((L,), jnp.int32)
    for g in range(PG):
        npeople = npeople + plsc.all_reduce_population_count(vaccs[g] > 0.0)
    total = jnp.sum(errs[0] + errs[1] + errs[2] + errs[3])
    return total / jnp.maximum(npeople.astype(jnp.float32), 1.0)


def _body(preds_hbm, gts_hbm, out_hbm,
          preds_v0, gts_v0, preds_v1, gts_v1, res_v, sem):
    wid = lax.axis_index("s") * NC + lax.axis_index("c")
    i0 = wid * IPW
    lane = lax.iota(jnp.int32, L)
    cp0 = pltpu.async_copy(preds_hbm.at[i0], preds_v0, sem)
    cg0 = pltpu.async_copy(gts_hbm.at[i0], gts_v0, sem)
    cp1 = pltpu.async_copy(preds_hbm.at[i0 + 1], preds_v1, sem)
    cg1 = pltpu.async_copy(gts_hbm.at[i0 + 1], gts_v1, sem)
    cp0.wait()
    cg0.wait()
    loss0 = _image_loss(gts_v0, preds_v0, lane)
    cp1.wait()
    cg1.wait()
    loss1 = _image_loss(gts_v1, preds_v1, lane)
    res = jnp.where(lane == 0, loss0, jnp.where(lane == 1, loss1, 0.0))
    res_v[...] = res
    pltpu.sync_copy(res_v, out_hbm.at[wid])


def kernel(preds, gts):
    gts_flat = gts.reshape(B, FLAT)
    mesh = plsc.VectorSubcoreMesh(core_axis_name="c", subcore_axis_name="s")
    f = pl.kernel(
        _body,
        mesh=mesh,
        out_type=jax.ShapeDtypeStruct((NW, L), jnp.float32),
        scratch_types=[
            pltpu.VMEM((N,), jnp.float32),
            pltpu.VMEM((FLAT,), jnp.float32),
            pltpu.VMEM((N,), jnp.float32),
            pltpu.VMEM((FLAT,), jnp.float32),
            pltpu.VMEM((L,), jnp.float32),
            pltpu.SemaphoreType.DMA,
        ],
        compiler_params=pltpu.CompilerParams(needs_layout_passes=False),
    )
    out2d = f(preds, gts_flat)
    return out2d[:, :IPW].reshape(B)
